# Initial kernel scaffold; baseline (speedup 1.0000x reference)
import jax
import jax.numpy as jnp
from jax.experimental import pallas as pl


def kernel(feat, edge_index, W1, b1, gamma, beta, W2, b2):
    raise NotImplementedError("write your pallas kernel here")


# speedup vs baseline: 7.5175x; 7.5175x over previous
import functools

import jax
import jax.numpy as jnp
from jax import lax
from jax.experimental import pallas as pl
from jax.experimental.pallas import tpu as pltpu
from jax.experimental.pallas import tpu_sc as plsc

_NC = 2
_NS = 16
_NW = _NC * _NS
_K = 80
_ZR = 80


def _tile_ids():
    c = lax.axis_index("c")
    s = lax.axis_index("s")
    return c, s, s * _NC + c


def _make_deg_kernel(E, NP):
    CT = (E // _K) // _NW
    RT = NP // _NS
    mesh = plsc.VectorSubcoreMesh(core_axis_name="c", subcore_axis_name="s")

    @functools.partial(
        pl.kernel,
        out_type=jax.ShapeDtypeStruct((_NC * 2 * NP,), jnp.float32),
        mesh=mesh,
        scratch_types=[
            pltpu.VMEM((CT, _K), jnp.int32),
            pltpu.VMEM((CT, _K), jnp.int32),
            pltpu.VMEM((_K,), jnp.float32),
            pltpu.VMEM((RT,), jnp.float32),
            pltpu.VMEM_SHARED((NP,), jnp.float32),
            pltpu.VMEM_SHARED((NP,), jnp.float32),
        ],
    )
    def deg_kernel(src_hbm, dst_hbm, deg_hbm, src_v, dst_v, ones_v, zero_v,
                   sdeg_s, sdeg_d):
        c, s, w = _tile_ids()

        @pl.loop(0, _K // 16)
        def _(i):
            ones_v[pl.ds(i * 16, 16)] = jnp.ones((16,), jnp.float32)

        @pl.loop(0, RT // 16)
        def _(i):
            zero_v[pl.ds(i * 16, 16)] = jnp.zeros((16,), jnp.float32)

        pltpu.sync_copy(zero_v, sdeg_s.at[pl.ds(s * RT, RT)])
        pltpu.sync_copy(zero_v, sdeg_d.at[pl.ds(s * RT, RT)])
        pltpu.sync_copy(src_hbm.at[w], src_v)
        pltpu.sync_copy(dst_hbm.at[w], dst_v)
        plsc.subcore_barrier()

        @pl.loop(0, CT)
        def _(i):
            pltpu.sync_copy(ones_v, sdeg_s.at[src_v.at[i]], add=True)
            pltpu.sync_copy(ones_v, sdeg_d.at[dst_v.at[i]], add=True)

        plsc.subcore_barrier()
        pltpu.sync_copy(sdeg_s.at[pl.ds(s * RT, RT)],
                        deg_hbm.at[pl.ds((c * 2 + 0) * NP + s * RT, RT)])
        pltpu.sync_copy(sdeg_d.at[pl.ds(s * RT, RT)],
                        deg_hbm.at[pl.ds((c * 2 + 1) * NP + s * RT, RT)])

    return deg_kernel


def _make_agg_kernel(N, D, E, NP):
    CT = (E // _K) // _NW
    RT = NP // _NS
    mesh = plsc.VectorSubcoreMesh(core_axis_name="c", subcore_axis_name="s")

    @functools.partial(
        pl.kernel,
        out_type=jax.ShapeDtypeStruct((_NC, NP, D), jnp.float32),
        mesh=mesh,
        scratch_types=[
            pltpu.VMEM((CT, _K), jnp.int32),
            pltpu.VMEM((CT, _K), jnp.int32),
            pltpu.VMEM((_K, D), jnp.float32),
            pltpu.VMEM_SHARED((NP, D), jnp.float32),
            pltpu.SemaphoreType.DMA,
        ],
    )
    def agg_kernel(h_hbm, src_hbm, dst_hbm, out_hbm, src_v, dst_v, rows_v,
                   agg_sh, sem):
        c, s, w = _tile_ids()

        @pl.loop(0, _ZR)
        def _(r):
            @pl.loop(0, D // 16)
            def _(j):
                rows_v[r, pl.ds(j * 16, 16)] = jnp.zeros((16,), jnp.float32)

        @pl.loop(0, RT // _ZR)
        def _(k):
            pltpu.sync_copy(rows_v, agg_sh.at[pl.ds(s * RT + k * _ZR, _ZR)])

        pltpu.sync_copy(src_hbm.at[w], src_v)
        pltpu.sync_copy(dst_hbm.at[w], dst_v)
        plsc.subcore_barrier()

        @pl.loop(0, CT)
        def _(i):
            pltpu.async_copy(h_hbm.at[src_v.at[i]], rows_v, sem).wait()
            pltpu.sync_copy(rows_v, agg_sh.at[dst_v.at[i]], add=True)

        plsc.subcore_barrier()
        pltpu.sync_copy(agg_sh.at[pl.ds(s * RT, RT)],
                        out_hbm.at[c, pl.ds(s * RT, RT)])

    return agg_kernel


def _norm_scale_body(feat_ref, ds0_ref, ds1_ref, dd0_ref, dd1_ref,
                     h0_ref, ns_ref, nd_ref):
    ns = lax.rsqrt(jnp.maximum(ds0_ref[...] + ds1_ref[...], 1.0))
    nd = lax.rsqrt(jnp.maximum(dd0_ref[...] + dd1_ref[...], 1.0))
    ns_ref[...] = ns
    nd_ref[...] = nd
    h0_ref[...] = feat_ref[...] * ns


def _dense_mid_body(q0_ref, q1_ref, nd_ref, ns_ref, w_ref, b_ref, g_ref,
                    be_ref, out_ref):
    x = (q0_ref[...] + q1_ref[...]) * nd_ref[...]
    y = jnp.dot(x, w_ref[...], preferred_element_type=jnp.float32) + b_ref[...]
    mu = jnp.mean(y, axis=-1, keepdims=True)
    var = jnp.mean((y - mu) ** 2, axis=-1, keepdims=True)
    y = (y - mu) * lax.rsqrt(var + 1e-5) * g_ref[...] + be_ref[...]
    out_ref[...] = jnp.maximum(y, 0.0) * ns_ref[...]


def _dense_out_body(q0_ref, q1_ref, nd_ref, w_ref, b_ref, out_ref):
    x = (q0_ref[...] + q1_ref[...]) * nd_ref[...]
    out_ref[...] = jnp.dot(x, w_ref[...],
                           preferred_element_type=jnp.float32) + b_ref[...]


def kernel(feat, edge_index, W1, b1, gamma, beta, W2, b2):
    N, D = feat.shape
    E = edge_index.shape[1]
    NP = -(-N // (_NS * _ZR)) * (_NS * _ZR)

    CT = (E // _K) // _NW
    src3d = edge_index[0].reshape(_NW, CT, _K)
    dst3d = edge_index[1].reshape(_NW, CT, _K)

    deg_kernel = _make_deg_kernel(E, NP)
    agg_kernel = _make_agg_kernel(N, D, E, NP)

    deg = deg_kernel(src3d, dst3d).reshape(_NC, 2, NP)
    ds0 = deg[0, 0, :N, None]
    ds1 = deg[1, 0, :N, None]
    dd0 = deg[0, 1, :N, None]
    dd1 = deg[1, 1, :N, None]

    BR = 2000
    grid = (N // BR,)
    vec_spec = pl.BlockSpec((BR, 1), lambda i: (i, 0))
    mat_spec = pl.BlockSpec((BR, D), lambda i: (i, 0))
    w_spec = pl.BlockSpec((D, D), lambda i: (0, 0))
    row_spec = pl.BlockSpec((1, D), lambda i: (0, 0))
    f32 = jnp.float32

    h0, ns, nd = pl.pallas_call(
        _norm_scale_body,
        grid=grid,
        in_specs=[mat_spec, vec_spec, vec_spec, vec_spec, vec_spec],
        out_specs=[mat_spec, vec_spec, vec_spec],
        out_shape=[jax.ShapeDtypeStruct((N, D), f32),
                   jax.ShapeDtypeStruct((N, 1), f32),
                   jax.ShapeDtypeStruct((N, 1), f32)],
    )(feat, ds0, ds1, dd0, dd1)

    agg1 = agg_kernel(h0, src3d, dst3d)

    h1 = pl.pallas_call(
        _dense_mid_body,
        grid=grid,
        in_specs=[mat_spec, mat_spec, vec_spec, vec_spec, w_spec, row_spec,
                  row_spec, row_spec],
        out_specs=mat_spec,
        out_shape=jax.ShapeDtypeStruct((N, D), f32),
    )(agg1[0, :N], agg1[1, :N], nd, ns, W1, b1.reshape(1, D),
      gamma.reshape(1, D), beta.reshape(1, D))

    agg2 = agg_kernel(h1, src3d, dst3d)

    out = pl.pallas_call(
        _dense_out_body,
        grid=grid,
        in_specs=[mat_spec, mat_spec, vec_spec, w_spec, row_spec],
        out_specs=mat_spec,
        out_shape=jax.ShapeDtypeStruct((N, D), f32),
    )(agg2[0, :N], agg2[1, :N], nd, W2, b2.reshape(1, D))

    return out


# speedup vs baseline: 10.9026x; 1.4503x over previous
import functools

import jax
import jax.numpy as jnp
from jax import lax
from jax.experimental import pallas as pl
from jax.experimental.pallas import tpu as pltpu
from jax.experimental.pallas import tpu_sc as plsc

_NC = 2
_NS = 16
_NW = _NC * _NS
_K = 80
_ZR = 80


def _tile_ids():
    c = lax.axis_index("c")
    s = lax.axis_index("s")
    return c, s, s * _NC + c


def _make_deg_kernel(E, NP):
    CT = (E // _K) // _NW
    RT = NP // _NS
    mesh = plsc.VectorSubcoreMesh(core_axis_name="c", subcore_axis_name="s")

    @functools.partial(
        pl.kernel,
        out_type=jax.ShapeDtypeStruct((_NC * 2 * NP,), jnp.float32),
        mesh=mesh,
        scratch_types=[
            pltpu.VMEM((CT, _K), jnp.int32),
            pltpu.VMEM((CT, _K), jnp.int32),
            pltpu.VMEM((_K,), jnp.float32),
            pltpu.VMEM((RT,), jnp.float32),
            pltpu.VMEM_SHARED((NP,), jnp.float32),
            pltpu.VMEM_SHARED((NP,), jnp.float32),
        ],
    )
    def deg_kernel(src_hbm, dst_hbm, deg_hbm, src_v, dst_v, ones_v, zero_v,
                   sdeg_s, sdeg_d):
        c, s, w = _tile_ids()

        @pl.loop(0, _K // 16)
        def _(i):
            ones_v[pl.ds(i * 16, 16)] = jnp.ones((16,), jnp.float32)

        @pl.loop(0, RT // 16)
        def _(i):
            zero_v[pl.ds(i * 16, 16)] = jnp.zeros((16,), jnp.float32)

        pltpu.sync_copy(zero_v, sdeg_s.at[pl.ds(s * RT, RT)])
        pltpu.sync_copy(zero_v, sdeg_d.at[pl.ds(s * RT, RT)])
        pltpu.sync_copy(src_hbm.at[w], src_v)
        pltpu.sync_copy(dst_hbm.at[w], dst_v)
        plsc.subcore_barrier()

        @pl.loop(0, CT)
        def _(i):
            pltpu.sync_copy(ones_v, sdeg_s.at[src_v.at[i]], add=True)
            pltpu.sync_copy(ones_v, sdeg_d.at[dst_v.at[i]], add=True)

        plsc.subcore_barrier()
        pltpu.sync_copy(sdeg_s.at[pl.ds(s * RT, RT)],
                        deg_hbm.at[pl.ds((c * 2 + 0) * NP + s * RT, RT)])
        pltpu.sync_copy(sdeg_d.at[pl.ds(s * RT, RT)],
                        deg_hbm.at[pl.ds((c * 2 + 1) * NP + s * RT, RT)])

    return deg_kernel


def _make_agg_kernel(N, D, E, NP):
    KA = 80
    B = 25
    NB = (E // KA) // _NW // B
    RT = NP // _NS
    mesh = plsc.VectorSubcoreMesh(core_axis_name="c", subcore_axis_name="s")

    @functools.partial(
        pl.kernel,
        out_type=jax.ShapeDtypeStruct((_NC, NP, D), jnp.float32),
        mesh=mesh,
        scratch_types=[
            pltpu.VMEM((2, B, KA), jnp.int32),
            pltpu.VMEM((2, B, KA), jnp.int32),
            pltpu.VMEM((KA, D), jnp.float32),
            pltpu.VMEM((KA, D), jnp.float32),
            pltpu.VMEM_SHARED((NP, D), jnp.float32),
            pltpu.SemaphoreType.DMA,
            pltpu.SemaphoreType.DMA,
            pltpu.SemaphoreType.DMA,
            pltpu.SemaphoreType.DMA,
        ],
    )
    def agg_kernel(h_hbm, src_hbm, dst_hbm, out_hbm, src_v, dst_v, rows0_v,
                   rows1_v, agg_sh, sem0, sem1, sem_is, sem_id):
        c, s, w = _tile_ids()

        @pl.loop(0, KA)
        def _(r):
            @pl.loop(0, D // 16)
            def _(j):
                rows0_v[r, pl.ds(j * 16, 16)] = jnp.zeros((16,), jnp.float32)

        @pl.loop(0, RT // KA)
        def _(k):
            pltpu.sync_copy(rows0_v, agg_sh.at[pl.ds(s * RT + k * KA, KA)])

        pltpu.sync_copy(src_hbm.at[w, 0], src_v.at[0])
        pltpu.sync_copy(dst_hbm.at[w, 0], dst_v.at[0])
        plsc.subcore_barrier()

        for bb in range(NB):
            sv = src_v.at[bb % 2]
            dv = dst_v.at[bb % 2]
            if bb + 1 < NB:
                pltpu.async_copy(src_hbm.at[w, bb + 1],
                                 src_v.at[(bb + 1) % 2], sem_is)
                pltpu.async_copy(dst_hbm.at[w, bb + 1],
                                 dst_v.at[(bb + 1) % 2], sem_id)

            pltpu.async_copy(h_hbm.at[sv.at[0]], rows0_v, sem0)
            pltpu.async_copy(h_hbm.at[sv.at[1]], rows1_v, sem1)

            @pl.loop(0, B - 1, step=2)
            def _(i):
                pltpu.make_async_copy(h_hbm.at[sv.at[i]], rows0_v,
                                      sem0).wait()
                pltpu.sync_copy(rows0_v, agg_sh.at[dv.at[i]], add=True)

                @pl.when(i + 2 < B)
                def _():
                    pltpu.async_copy(h_hbm.at[sv.at[i + 2]], rows0_v, sem0)

                pltpu.make_async_copy(h_hbm.at[sv.at[i + 1]], rows1_v,
                                      sem1).wait()
                pltpu.sync_copy(rows1_v, agg_sh.at[dv.at[i + 1]], add=True)

                @pl.when(i + 3 < B)
                def _():
                    pltpu.async_copy(h_hbm.at[sv.at[i + 3]], rows1_v, sem1)

            pltpu.make_async_copy(h_hbm.at[sv.at[B - 1]], rows0_v,
                                  sem0).wait()
            pltpu.sync_copy(rows0_v, agg_sh.at[dv.at[B - 1]], add=True)

            if bb + 1 < NB:
                pltpu.make_async_copy(src_hbm.at[w, bb + 1],
                                      src_v.at[(bb + 1) % 2], sem_is).wait()
                pltpu.make_async_copy(dst_hbm.at[w, bb + 1],
                                      dst_v.at[(bb + 1) % 2], sem_id).wait()

        plsc.subcore_barrier()
        pltpu.sync_copy(agg_sh.at[pl.ds(s * RT, RT)],
                        out_hbm.at[c, pl.ds(s * RT, RT)])

    return agg_kernel


def _norm_scale_body(feat_ref, ds0_ref, ds1_ref, dd0_ref, dd1_ref,
                     h0_ref, ns_ref, nd_ref):
    ns = lax.rsqrt(jnp.maximum(ds0_ref[...] + ds1_ref[...], 1.0))
    nd = lax.rsqrt(jnp.maximum(dd0_ref[...] + dd1_ref[...], 1.0))
    ns_ref[...] = ns
    nd_ref[...] = nd
    h0_ref[...] = feat_ref[...] * ns


def _dense_mid_body(q0_ref, q1_ref, nd_ref, ns_ref, w_ref, b_ref, g_ref,
                    be_ref, out_ref):
    x = (q0_ref[...] + q1_ref[...]) * nd_ref[...]
    y = jnp.dot(x, w_ref[...], preferred_element_type=jnp.float32) + b_ref[...]
    mu = jnp.mean(y, axis=-1, keepdims=True)
    var = jnp.mean((y - mu) ** 2, axis=-1, keepdims=True)
    y = (y - mu) * lax.rsqrt(var + 1e-5) * g_ref[...] + be_ref[...]
    out_ref[...] = jnp.maximum(y, 0.0) * ns_ref[...]


def _dense_out_body(q0_ref, q1_ref, nd_ref, w_ref, b_ref, out_ref):
    x = (q0_ref[...] + q1_ref[...]) * nd_ref[...]
    out_ref[...] = jnp.dot(x, w_ref[...],
                           preferred_element_type=jnp.float32) + b_ref[...]


def kernel(feat, edge_index, W1, b1, gamma, beta, W2, b2):
    N, D = feat.shape
    E = edge_index.shape[1]
    NP = -(-N // (_NS * _ZR)) * (_NS * _ZR)

    src3d = edge_index[0].reshape(_NW, (E // _K) // _NW, _K)
    dst3d = edge_index[1].reshape(_NW, (E // _K) // _NW, _K)
    NB = (E // _K) // _NW // 25
    src3da = edge_index[0].reshape(_NW, NB, 25, _K)
    dst3da = edge_index[1].reshape(_NW, NB, 25, _K)

    deg_kernel = _make_deg_kernel(E, NP)
    agg_kernel = _make_agg_kernel(N, D, E, NP)

    deg = deg_kernel(src3d, dst3d).reshape(_NC, 2, NP)
    ds0 = deg[0, 0, :N, None]
    ds1 = deg[1, 0, :N, None]
    dd0 = deg[0, 1, :N, None]
    dd1 = deg[1, 1, :N, None]

    BR = 2000
    grid = (N // BR,)
    vec_spec = pl.BlockSpec((BR, 1), lambda i: (i, 0))
    mat_spec = pl.BlockSpec((BR, D), lambda i: (i, 0))
    w_spec = pl.BlockSpec((D, D), lambda i: (0, 0))
    row_spec = pl.BlockSpec((1, D), lambda i: (0, 0))
    f32 = jnp.float32

    h0, ns, nd = pl.pallas_call(
        _norm_scale_body,
        grid=grid,
        in_specs=[mat_spec, vec_spec, vec_spec, vec_spec, vec_spec],
        out_specs=[mat_spec, vec_spec, vec_spec],
        out_shape=[jax.ShapeDtypeStruct((N, D), f32),
                   jax.ShapeDtypeStruct((N, 1), f32),
                   jax.ShapeDtypeStruct((N, 1), f32)],
    )(feat, ds0, ds1, dd0, dd1)

    agg1 = agg_kernel(h0, src3da, dst3da)

    h1 = pl.pallas_call(
        _dense_mid_body,
        grid=grid,
        in_specs=[mat_spec, mat_spec, vec_spec, vec_spec, w_spec, row_spec,
                  row_spec, row_spec],
        out_specs=mat_spec,
        out_shape=jax.ShapeDtypeStruct((N, D), f32),
    )(agg1[0, :N], agg1[1, :N], nd, ns, W1, b1.reshape(1, D),
      gamma.reshape(1, D), beta.reshape(1, D))

    agg2 = agg_kernel(h1, src3da, dst3da)

    out = pl.pallas_call(
        _dense_out_body,
        grid=grid,
        in_specs=[mat_spec, mat_spec, vec_spec, w_spec, row_spec],
        out_specs=mat_spec,
        out_shape=jax.ShapeDtypeStruct((N, D), f32),
    )(agg2[0, :N], agg2[1, :N], nd, W2, b2.reshape(1, D))

    return out


# speedup vs baseline: 11.5717x; 1.0614x over previous
import functools

import jax
import jax.numpy as jnp
from jax import lax
from jax.experimental import pallas as pl
from jax.experimental.pallas import tpu as pltpu
from jax.experimental.pallas import tpu_sc as plsc

_NC = 2
_NS = 16
_NW = _NC * _NS
_K = 80
_ZR = 80


def _tile_ids():
    c = lax.axis_index("c")
    s = lax.axis_index("s")
    return c, s, s * _NC + c


def _make_deg_kernel(E, NP):
    CT = (E // _K) // _NW
    RT = NP // _NS
    mesh = plsc.VectorSubcoreMesh(core_axis_name="c", subcore_axis_name="s")

    @functools.partial(
        pl.kernel,
        out_type=jax.ShapeDtypeStruct((_NC * 2 * NP,), jnp.float32),
        mesh=mesh,
        scratch_types=[
            pltpu.VMEM((CT, _K), jnp.int32),
            pltpu.VMEM((CT, _K), jnp.int32),
            pltpu.VMEM((_K,), jnp.float32),
            pltpu.VMEM((RT,), jnp.float32),
            pltpu.VMEM_SHARED((NP,), jnp.float32),
            pltpu.VMEM_SHARED((NP,), jnp.float32),
        ],
    )
    def deg_kernel(src_hbm, dst_hbm, deg_hbm, src_v, dst_v, ones_v, zero_v,
                   sdeg_s, sdeg_d):
        c, s, w = _tile_ids()

        @pl.loop(0, _K // 16)
        def _(i):
            ones_v[pl.ds(i * 16, 16)] = jnp.ones((16,), jnp.float32)

        @pl.loop(0, RT // 16)
        def _(i):
            zero_v[pl.ds(i * 16, 16)] = jnp.zeros((16,), jnp.float32)

        pltpu.sync_copy(zero_v, sdeg_s.at[pl.ds(s * RT, RT)])
        pltpu.sync_copy(zero_v, sdeg_d.at[pl.ds(s * RT, RT)])
        pltpu.sync_copy(src_hbm.at[w], src_v)
        pltpu.sync_copy(dst_hbm.at[w], dst_v)
        plsc.subcore_barrier()

        @pl.loop(0, CT)
        def _(i):
            pltpu.sync_copy(ones_v, sdeg_s.at[src_v.at[i]], add=True)
            pltpu.sync_copy(ones_v, sdeg_d.at[dst_v.at[i]], add=True)

        plsc.subcore_barrier()
        pltpu.sync_copy(sdeg_s.at[pl.ds(s * RT, RT)],
                        deg_hbm.at[pl.ds((c * 2 + 0) * NP + s * RT, RT)])
        pltpu.sync_copy(sdeg_d.at[pl.ds(s * RT, RT)],
                        deg_hbm.at[pl.ds((c * 2 + 1) * NP + s * RT, RT)])

    return deg_kernel


def _make_agg_kernel(N, D, Epad, NP):
    KA = 128
    B = 20
    NB = (Epad // KA) // _NW // B
    RT = NP // _NS
    mesh = plsc.VectorSubcoreMesh(core_axis_name="c", subcore_axis_name="s")

    @functools.partial(
        pl.kernel,
        out_type=jax.ShapeDtypeStruct((_NC, NP, D), jnp.float32),
        mesh=mesh,
        scratch_types=[
            pltpu.VMEM((2, B, KA), jnp.int32),
            pltpu.VMEM((2, B, KA), jnp.int32),
            pltpu.VMEM((KA, D), jnp.float32),
            pltpu.VMEM((KA, D), jnp.float32),
            pltpu.VMEM_SHARED((NP, D), jnp.float32),
            pltpu.SemaphoreType.DMA,
            pltpu.SemaphoreType.DMA,
            pltpu.SemaphoreType.DMA,
            pltpu.SemaphoreType.DMA,
        ],
    )
    def agg_kernel(h_hbm, src_hbm, dst_hbm, out_hbm, src_v, dst_v, rows0_v,
                   rows1_v, agg_sh, sem0, sem1, sem_is, sem_id):
        c, s, w = _tile_ids()

        @pl.loop(0, KA)
        def _(r):
            @pl.loop(0, D // 16)
            def _(j):
                rows0_v[r, pl.ds(j * 16, 16)] = jnp.zeros((16,), jnp.float32)

        @pl.loop(0, RT // KA)
        def _(k):
            pltpu.sync_copy(rows0_v, agg_sh.at[pl.ds(s * RT + k * KA, KA)])

        pltpu.sync_copy(src_hbm.at[w, 0], src_v.at[0])
        pltpu.sync_copy(dst_hbm.at[w, 0], dst_v.at[0])
        plsc.subcore_barrier()

        pltpu.async_copy(h_hbm.at[src_v.at[0].at[0]], rows0_v, sem0)
        pltpu.async_copy(h_hbm.at[src_v.at[0].at[1]], rows1_v, sem1)
        for bb in range(NB):
            sv = src_v.at[bb % 2]
            dv = dst_v.at[bb % 2]
            if bb + 1 < NB:
                pltpu.async_copy(src_hbm.at[w, bb + 1],
                                 src_v.at[(bb + 1) % 2], sem_is)
                pltpu.async_copy(dst_hbm.at[w, bb + 1],
                                 dst_v.at[(bb + 1) % 2], sem_id)

            @pl.loop(0, B, step=2)
            def _(i):
                pltpu.make_async_copy(h_hbm.at[sv.at[i]], rows0_v,
                                      sem0).wait()
                pltpu.sync_copy(rows0_v, agg_sh.at[dv.at[i]], add=True)

                @pl.when(i + 2 < B)
                def _():
                    pltpu.async_copy(h_hbm.at[sv.at[i + 2]], rows0_v, sem0)

                pltpu.make_async_copy(h_hbm.at[sv.at[i + 1]], rows1_v,
                                      sem1).wait()
                pltpu.sync_copy(rows1_v, agg_sh.at[dv.at[i + 1]], add=True)

                @pl.when(i + 3 < B)
                def _():
                    pltpu.async_copy(h_hbm.at[sv.at[i + 3]], rows1_v, sem1)

            if bb + 1 < NB:
                nsv = src_v.at[(bb + 1) % 2]
                pltpu.make_async_copy(src_hbm.at[w, bb + 1], nsv,
                                      sem_is).wait()
                pltpu.make_async_copy(dst_hbm.at[w, bb + 1],
                                      dst_v.at[(bb + 1) % 2], sem_id).wait()
                pltpu.async_copy(h_hbm.at[nsv.at[0]], rows0_v, sem0)
                pltpu.async_copy(h_hbm.at[nsv.at[1]], rows1_v, sem1)

        plsc.subcore_barrier()
        pltpu.sync_copy(agg_sh.at[pl.ds(s * RT, RT)],
                        out_hbm.at[c, pl.ds(s * RT, RT)])

    return agg_kernel


def _norm_scale_body(feat_ref, ds0_ref, ds1_ref, dd0_ref, dd1_ref,
                     h0_ref, ns_ref, nd_ref):
    ns = lax.rsqrt(jnp.maximum(ds0_ref[...] + ds1_ref[...], 1.0))
    nd = lax.rsqrt(jnp.maximum(dd0_ref[...] + dd1_ref[...], 1.0))
    ns_ref[...] = ns
    nd_ref[...] = nd
    h0_ref[...] = feat_ref[...] * ns


def _dense_mid_body(q0_ref, q1_ref, nd_ref, ns_ref, w_ref, b_ref, g_ref,
                    be_ref, out_ref):
    x = (q0_ref[...] + q1_ref[...]) * nd_ref[...]
    y = jnp.dot(x, w_ref[...], preferred_element_type=jnp.float32) + b_ref[...]
    mu = jnp.mean(y, axis=-1, keepdims=True)
    var = jnp.mean((y - mu) ** 2, axis=-1, keepdims=True)
    y = (y - mu) * lax.rsqrt(var + 1e-5) * g_ref[...] + be_ref[...]
    out_ref[...] = jnp.maximum(y, 0.0) * ns_ref[...]


def _dense_out_body(q0_ref, q1_ref, nd_ref, w_ref, b_ref, out_ref):
    x = (q0_ref[...] + q1_ref[...]) * nd_ref[...]
    out_ref[...] = jnp.dot(x, w_ref[...],
                           preferred_element_type=jnp.float32) + b_ref[...]


def kernel(feat, edge_index, W1, b1, gamma, beta, W2, b2):
    N, D = feat.shape
    E = edge_index.shape[1]
    NP = -(-N // (_NS * _ZR)) * (_NS * _ZR)

    src3d = edge_index[0].reshape(_NW, (E // _K) // _NW, _K)
    dst3d = edge_index[1].reshape(_NW, (E // _K) // _NW, _K)

    per_tile = -(-(E // _NW) // 2560) * 2560
    Epad = per_tile * _NW
    pad = Epad - E
    pad_src = (jnp.arange(pad, dtype=jnp.int32) * 97) % N
    pad_dst = N + (jnp.arange(pad, dtype=jnp.int32) % (NP - N))
    srcp = jnp.concatenate([edge_index[0], pad_src])
    dstp = jnp.concatenate([edge_index[1], pad_dst])
    NB, B, KA = per_tile // 2560, 20, 128
    src3da = srcp.reshape(_NW, NB, B, KA)
    dst3da = dstp.reshape(_NW, NB, B, KA)

    deg_kernel = _make_deg_kernel(E, NP)
    agg_kernel = _make_agg_kernel(N, D, Epad, NP)

    deg = deg_kernel(src3d, dst3d).reshape(_NC, 2, NP)
    ds0 = deg[0, 0, :N, None]
    ds1 = deg[1, 0, :N, None]
    dd0 = deg[0, 1, :N, None]
    dd1 = deg[1, 1, :N, None]

    BR = 2000
    grid = (N // BR,)
    vec_spec = pl.BlockSpec((BR, 1), lambda i: (i, 0))
    mat_spec = pl.BlockSpec((BR, D), lambda i: (i, 0))
    w_spec = pl.BlockSpec((D, D), lambda i: (0, 0))
    row_spec = pl.BlockSpec((1, D), lambda i: (0, 0))
    f32 = jnp.float32

    h0, ns, nd = pl.pallas_call(
        _norm_scale_body,
        grid=grid,
        in_specs=[mat_spec, vec_spec, vec_spec, vec_spec, vec_spec],
        out_specs=[mat_spec, vec_spec, vec_spec],
        out_shape=[jax.ShapeDtypeStruct((N, D), f32),
                   jax.ShapeDtypeStruct((N, 1), f32),
                   jax.ShapeDtypeStruct((N, 1), f32)],
    )(feat, ds0, ds1, dd0, dd1)

    agg1 = agg_kernel(h0, src3da, dst3da)

    h1 = pl.pallas_call(
        _dense_mid_body,
        grid=grid,
        in_specs=[mat_spec, mat_spec, vec_spec, vec_spec, w_spec, row_spec,
                  row_spec, row_spec],
        out_specs=mat_spec,
        out_shape=jax.ShapeDtypeStruct((N, D), f32),
    )(agg1[0, :N], agg1[1, :N], nd, ns, W1, b1.reshape(1, D),
      gamma.reshape(1, D), beta.reshape(1, D))

    agg2 = agg_kernel(h1, src3da, dst3da)

    out = pl.pallas_call(
        _dense_out_body,
        grid=grid,
        in_specs=[mat_spec, mat_spec, vec_spec, w_spec, row_spec],
        out_specs=mat_spec,
        out_shape=jax.ShapeDtypeStruct((N, D), f32),
    )(agg2[0, :N], agg2[1, :N], nd, W2, b2.reshape(1, D))

    return out
